# baseline (speedup 1.0000x reference)
import jax
import jax.numpy as jnp
from jax import lax
from jax.experimental import pallas as pl
from jax.experimental.pallas import tpu as pltpu
from jax.experimental.pallas import tpu_sc as plsc

_K = 256
_RB = 8
_V = 100000
_CHUNK = 10000
_NCHUNK = _V // _CHUNK
_NTILES = 32
_RPT = 128 // _NTILES
_L = 16


def _sortable(b):
    return b ^ ((b >> 31) & jnp.int32(0x7FFFFFFF))


def _sc_body(pb_hbm, pd_hbm, lik_hbm, out_hbm,
             l0, l1, b0, b1, d0, d1, hist_b, hist_d, obuf,
             sl0, sl1, sb0, sb1, sd0, sd1):
    wid = lax.axis_index("s") * 2 + lax.axis_index("c")
    lane = lax.iota(jnp.int32, _L)
    ones = jnp.ones((_L,), jnp.int32)
    zero = jnp.zeros((_L,), jnp.int32)
    slots = ((l0, b0, d0), (l1, b1, d1))
    sems = ((sl0, sb0, sd0), (sl1, sb1, sd1))

    def start(row, c, slot):
        off = row * _V + c * _CHUNK
        lr, br, dr = slots[slot]
        ls, bs, ds_ = sems[slot]
        return (
            pltpu.async_copy(lik_hbm.at[pl.ds(off, _CHUNK)], lr, ls),
            pltpu.async_copy(pb_hbm.at[pl.ds(off, _CHUNK)], br, bs),
            pltpu.async_copy(pd_hbm.at[pl.ds(off, _CHUNK)], dr, ds_),
        )

    def zero_hists():
        def zb(j, c):
            hist_b[pl.ds(j * _L, _L)] = zero
            hist_d[pl.ds(j * _L, _L)] = zero
            return c
        lax.fori_loop(0, 4096 // _L, zb, 0)

    def chunk_loop(slot, fn, carry):
        lr, br, dr = slots[slot]

        def body(i, car):
            sl = pl.ds(i * _L, _L)
            l = lr[sl]
            xb = br[sl] + l
            xd = dr[sl] + l
            sb = _sortable(lax.bitcast_convert_type(xb, jnp.int32))
            sd = _sortable(lax.bitcast_convert_type(xd, jnp.int32))
            return fn(xb, xd, sb, sd, car)

        return lax.fori_loop(0, _CHUNK // _L, body, carry)

    def stream_pass(row, fn, carry):
        hs = start(row, 0, 0)
        for c in range(_NCHUNK):
            nxt = start(row, c + 1, (c + 1) & 1) if c + 1 < _NCHUNK else None
            for h in hs:
                h.wait()
            carry = chunk_loop(c & 1, fn, carry)
            hs = nxt
        return carry

    def scan_hist(hist_ref, nvregs, running0):
        def body(j, car):
            running, p, c_hi, found = car
            base = (nvregs - 1 - j) * _L
            v = hist_ref[pl.ds(base, _L)]
            rv = lax.rev(v, (0,))
            cs = plsc.cumsum(rv)
            tot = running + cs
            pred = tot >= _K
            npred = plsc.all_reduce_population_count(pred)
            f = plsc.all_reduce_ffs(pred)
            hit = (npred > 0) & jnp.logical_not(found)
            p_new = base + 15 - f
            c_hi_new = running + jnp.sum(jnp.where(lane < f, rv, zero))
            run_next = running + jnp.sum(v)
            return (run_next,
                    jnp.where(hit, p_new, p),
                    jnp.where(hit, c_hi_new, c_hi),
                    found | (npred > 0))

        init = (running0, zero, zero, jnp.zeros((_L,), jnp.bool_))
        _, p, c_hi, _ = lax.fori_loop(0, nvregs, body, init)
        return p, c_hi

    def row_task(rr, c0):
        row = wid * _RPT + rr

        zero_hists()

        def fn0(xb, xd, sb, sd, car):
            mb, md = car
            plsc.addupdate_scatter(hist_b, [(sb >> 20) + 2048], ones)
            plsc.addupdate_scatter(hist_d, [(sd >> 20) + 2048], ones)
            return (jnp.maximum(mb, xb), jnp.maximum(md, xd))

        ninf = jnp.full((_L,), -jnp.inf, jnp.float32)
        mb, md = stream_pass(row, fn0, (ninf, ninf))
        p0b, chib = scan_hist(hist_b, 4096 // _L, zero)
        p0d, chid = scan_hist(hist_d, 4096 // _L, zero)

        zero_hists()

        def fn1(xb, xd, sb, sd, car):
            plsc.addupdate_scatter(hist_b, [(sb >> 8) & 0xFFF], ones,
                                   mask=((sb >> 20) + 2048) == p0b)
            plsc.addupdate_scatter(hist_d, [(sd >> 8) & 0xFFF], ones,
                                   mask=((sd >> 20) + 2048) == p0d)
            return car

        stream_pass(row, fn1, 0)
        p1b, chib = scan_hist(hist_b, 4096 // _L, chib)
        p1d, chid = scan_hist(hist_d, 4096 // _L, chid)

        zero_hists()
        pre_b = ((p0b - 2048) << 12) + p1b
        pre_d = ((p0d - 2048) << 12) + p1d

        def fn2(xb, xd, sb, sd, car):
            plsc.addupdate_scatter(hist_b, [sb & 0xFF], ones,
                                   mask=(sb >> 8) == pre_b)
            plsc.addupdate_scatter(hist_d, [sd & 0xFF], ones,
                                   mask=(sd >> 8) == pre_d)
            return car

        stream_pass(row, fn2, 0)
        p2b, _ = scan_hist(hist_b, 256 // _L, chib)
        p2d, _ = scan_hist(hist_d, 256 // _L, chid)

        s_b = ((p0b - 2048) << 20) + (p1b << 8) + p2b
        s_d = ((p0d - 2048) << 20) + (p1d << 8) + p2d
        tfb = lax.bitcast_convert_type(_sortable(s_b), jnp.float32)
        tfd = lax.bitcast_convert_type(_sortable(s_d), jnp.float32)
        mfb = jnp.zeros((_L,), jnp.float32) + jnp.max(mb)
        mfd = jnp.zeros((_L,), jnp.float32) + jnp.max(md)

        cur = obuf[...]
        cur = jnp.where(lane == rr, tfb, cur)
        cur = jnp.where(lane == 4 + rr, tfd, cur)
        cur = jnp.where(lane == 8 + rr, mfb, cur)
        cur = jnp.where(lane == 12 + rr, mfd, cur)
        obuf[...] = cur
        return c0

    lax.fori_loop(0, _RPT, row_task, 0)
    pltpu.sync_copy(obuf, out_hbm.at[wid])


def _sc_stats(pb, pd, lik):
    mesh = plsc.VectorSubcoreMesh(core_axis_name="c", subcore_axis_name="s",
                                  num_cores=2, num_subcores=16)
    f32 = jnp.float32
    return pl.kernel(
        _sc_body,
        out_type=jax.ShapeDtypeStruct((_NTILES, _L), f32),
        mesh=mesh,
        compiler_params=pltpu.CompilerParams(needs_layout_passes=False),
        scratch_types=[
            pltpu.VMEM((_CHUNK,), f32), pltpu.VMEM((_CHUNK,), f32),
            pltpu.VMEM((_CHUNK,), f32), pltpu.VMEM((_CHUNK,), f32),
            pltpu.VMEM((_CHUNK,), f32), pltpu.VMEM((_CHUNK,), f32),
            pltpu.VMEM((4096,), jnp.int32), pltpu.VMEM((4096,), jnp.int32),
            pltpu.VMEM((_L,), f32),
            pltpu.SemaphoreType.DMA, pltpu.SemaphoreType.DMA,
            pltpu.SemaphoreType.DMA, pltpu.SemaphoreType.DMA,
            pltpu.SemaphoreType.DMA, pltpu.SemaphoreType.DMA,
        ],
    )(pb, pd, lik)


def _tc_body(pb_ref, pd_ref, lik_ref, st_ref, out_ref):
    lik = lik_ref[...]
    st = st_ref[...]
    for src, p_ref in ((0, pb_ref), (1, pd_ref)):
        x = p_ref[...] + lik
        t_f = st[:, src:src + 1]
        m_f = st[:, 2 + src:3 + src]
        e = jnp.where(x >= t_f, jnp.exp(x - m_f), jnp.float32(0.0))
        denom = jnp.sum(e, axis=-1, keepdims=True)
        out_ref[src] = e * (jnp.float32(1.0) / denom)


def kernel(prior_bass_logits, prior_drums_logits, likelihood_logits, top_k):
    del top_k
    B, V = prior_bass_logits.shape
    stats = _sc_stats(prior_bass_logits.reshape(-1),
                      prior_drums_logits.reshape(-1),
                      likelihood_logits.reshape(-1))
    st = stats.reshape(_NTILES, 4, _RPT).transpose(0, 2, 1).reshape(B, 4)
    in_spec = pl.BlockSpec((_RB, V), lambda i: (i, 0))
    return pl.pallas_call(
        _tc_body,
        grid=(B // _RB,),
        in_specs=[in_spec, in_spec, in_spec,
                  pl.BlockSpec((_RB, 4), lambda i: (i, 0))],
        out_specs=pl.BlockSpec((2, _RB, V), lambda i: (0, i, 0)),
        out_shape=jax.ShapeDtypeStruct((2, B, V), jnp.float32),
    )(prior_bass_logits, prior_drums_logits, likelihood_logits, st)


# speedup vs baseline: 1.7747x; 1.7747x over previous
import jax
import jax.numpy as jnp
from jax import lax
from jax.experimental import pallas as pl
from jax.experimental.pallas import tpu as pltpu
from jax.experimental.pallas import tpu_sc as plsc

_K = 256
_RB = 8
_V = 100000
_CHUNK = 10000
_NCHUNK = _V // _CHUNK
_NTILES = 32
_RPT = 128 // _NTILES
_L = 16


def _sortable(b):
    return b ^ ((b >> 31) & jnp.int32(0x7FFFFFFF))


def _sc_body(pb_hbm, pd_hbm, lik_hbm, out_hbm,
             l0, l1, b0, b1, d0, d1, hist_b, hist_d, obuf,
             sl0, sl1, sb0, sb1, sd0, sd1):
    wid = lax.axis_index("s") * 2 + lax.axis_index("c")
    lane = lax.iota(jnp.int32, _L)
    ones = jnp.ones((_L,), jnp.int32)
    zero = jnp.zeros((_L,), jnp.int32)
    slots = ((l0, b0, d0), (l1, b1, d1))
    sems = ((sl0, sb0, sd0), (sl1, sb1, sd1))

    def start(row, c, slot):
        off = row * _V + c * _CHUNK
        lr, br, dr = slots[slot]
        ls, bs, ds_ = sems[slot]
        return (
            pltpu.async_copy(lik_hbm.at[pl.ds(off, _CHUNK)], lr, ls),
            pltpu.async_copy(pb_hbm.at[pl.ds(off, _CHUNK)], br, bs),
            pltpu.async_copy(pd_hbm.at[pl.ds(off, _CHUNK)], dr, ds_),
        )

    def zero_hists():
        @plsc.parallel_loop(0, 4096 // _L, 1, unroll=8)
        def _(j):
            hist_b[pl.ds(j * _L, _L)] = zero
            hist_d[pl.ds(j * _L, _L)] = zero

    def chunk_loop(slot, fn, carry):
        lr, br, dr = slots[slot]

        def body(i, car):
            sl = pl.ds(i * _L, _L)
            l = lr[sl]
            xb = br[sl] + l
            xd = dr[sl] + l
            sb = _sortable(lax.bitcast_convert_type(xb, jnp.int32))
            sd = _sortable(lax.bitcast_convert_type(xd, jnp.int32))
            return fn(xb, xd, sb, sd, car)

        return plsc.parallel_loop(0, _CHUNK // _L, 1, unroll=5,
                                  carry=carry)(body)

    def stream_pass(row, fn, carry):
        hs = start(row, 0, 0)
        for c in range(_NCHUNK):
            nxt = start(row, c + 1, (c + 1) & 1) if c + 1 < _NCHUNK else None
            for h in hs:
                h.wait()
            carry = chunk_loop(c & 1, fn, carry)
            hs = nxt
        return carry

    def scan_hist(hist_ref, nvregs, running0):
        def body(j, car):
            running, p, c_hi, found = car
            base = (nvregs - 1 - j) * _L
            v = hist_ref[pl.ds(base, _L)]
            rv = lax.rev(v, (0,))
            cs = plsc.cumsum(rv)
            tot = running + cs
            pred = tot >= _K
            npred = plsc.all_reduce_population_count(pred)
            f = plsc.all_reduce_ffs(pred)
            hit = (npred > 0) & jnp.logical_not(found)
            p_new = base + 15 - f
            c_hi_new = running + jnp.sum(jnp.where(lane < f, rv, zero))
            run_next = running + jnp.sum(v)
            return (run_next,
                    jnp.where(hit, p_new, p),
                    jnp.where(hit, c_hi_new, c_hi),
                    found | (npred > 0))

        init = (running0, zero, zero, jnp.zeros((_L,), jnp.bool_))
        _, p, c_hi, _ = plsc.parallel_loop(0, nvregs, 1, unroll=4,
                                           carry=init)(body)
        return p, c_hi

    def row_task(rr, c0):
        row = wid * _RPT + rr

        zero_hists()

        def fn0(xb, xd, sb, sd, car):
            mb, md = car
            plsc.addupdate_scatter(hist_b, [(sb >> 20) + 2048], ones)
            plsc.addupdate_scatter(hist_d, [(sd >> 20) + 2048], ones)
            return (jnp.maximum(mb, xb), jnp.maximum(md, xd))

        ninf = jnp.full((_L,), -jnp.inf, jnp.float32)
        mb, md = stream_pass(row, fn0, (ninf, ninf))
        p0b, chib = scan_hist(hist_b, 4096 // _L, zero)
        p0d, chid = scan_hist(hist_d, 4096 // _L, zero)

        zero_hists()

        def fn1(xb, xd, sb, sd, car):
            plsc.addupdate_scatter(hist_b, [(sb >> 8) & 0xFFF], ones,
                                   mask=((sb >> 20) + 2048) == p0b)
            plsc.addupdate_scatter(hist_d, [(sd >> 8) & 0xFFF], ones,
                                   mask=((sd >> 20) + 2048) == p0d)
            return car

        stream_pass(row, fn1, jnp.int32(0))
        p1b, chib = scan_hist(hist_b, 4096 // _L, chib)
        p1d, chid = scan_hist(hist_d, 4096 // _L, chid)

        zero_hists()
        pre_b = ((p0b - 2048) << 12) + p1b
        pre_d = ((p0d - 2048) << 12) + p1d

        def fn2(xb, xd, sb, sd, car):
            plsc.addupdate_scatter(hist_b, [sb & 0xFF], ones,
                                   mask=(sb >> 8) == pre_b)
            plsc.addupdate_scatter(hist_d, [sd & 0xFF], ones,
                                   mask=(sd >> 8) == pre_d)
            return car

        stream_pass(row, fn2, jnp.int32(0))
        p2b, _ = scan_hist(hist_b, 256 // _L, chib)
        p2d, _ = scan_hist(hist_d, 256 // _L, chid)

        s_b = ((p0b - 2048) << 20) + (p1b << 8) + p2b
        s_d = ((p0d - 2048) << 20) + (p1d << 8) + p2d
        tfb = lax.bitcast_convert_type(_sortable(s_b), jnp.float32)
        tfd = lax.bitcast_convert_type(_sortable(s_d), jnp.float32)
        mfb = jnp.zeros((_L,), jnp.float32) + jnp.max(mb)
        mfd = jnp.zeros((_L,), jnp.float32) + jnp.max(md)

        cur = obuf[...]
        cur = jnp.where(lane == rr, tfb, cur)
        cur = jnp.where(lane == 4 + rr, tfd, cur)
        cur = jnp.where(lane == 8 + rr, mfb, cur)
        cur = jnp.where(lane == 12 + rr, mfd, cur)
        obuf[...] = cur
        return c0

    lax.fori_loop(0, _RPT, row_task, 0)
    pltpu.sync_copy(obuf, out_hbm.at[wid])


def _sc_stats(pb, pd, lik):
    mesh = plsc.VectorSubcoreMesh(core_axis_name="c", subcore_axis_name="s",
                                  num_cores=2, num_subcores=16)
    f32 = jnp.float32
    return pl.kernel(
        _sc_body,
        out_type=jax.ShapeDtypeStruct((_NTILES, _L), f32),
        mesh=mesh,
        compiler_params=pltpu.CompilerParams(needs_layout_passes=False),
        scratch_types=[
            pltpu.VMEM((_CHUNK,), f32), pltpu.VMEM((_CHUNK,), f32),
            pltpu.VMEM((_CHUNK,), f32), pltpu.VMEM((_CHUNK,), f32),
            pltpu.VMEM((_CHUNK,), f32), pltpu.VMEM((_CHUNK,), f32),
            pltpu.VMEM((4096,), jnp.int32), pltpu.VMEM((4096,), jnp.int32),
            pltpu.VMEM((_L,), f32),
            pltpu.SemaphoreType.DMA, pltpu.SemaphoreType.DMA,
            pltpu.SemaphoreType.DMA, pltpu.SemaphoreType.DMA,
            pltpu.SemaphoreType.DMA, pltpu.SemaphoreType.DMA,
        ],
    )(pb, pd, lik)


def _tc_body(pb_ref, pd_ref, lik_ref, st_ref, out_ref):
    lik = lik_ref[...]
    st = st_ref[...]
    for src, p_ref in ((0, pb_ref), (1, pd_ref)):
        x = p_ref[...] + lik
        t_f = st[:, src:src + 1]
        m_f = st[:, 2 + src:3 + src]
        e = jnp.where(x >= t_f, jnp.exp(x - m_f), jnp.float32(0.0))
        denom = jnp.sum(e, axis=-1, keepdims=True)
        out_ref[src] = e * (jnp.float32(1.0) / denom)


def kernel(prior_bass_logits, prior_drums_logits, likelihood_logits, top_k):
    del top_k
    B, V = prior_bass_logits.shape
    stats = _sc_stats(prior_bass_logits.reshape(-1),
                      prior_drums_logits.reshape(-1),
                      likelihood_logits.reshape(-1))
    st = stats.reshape(_NTILES, 4, _RPT).transpose(0, 2, 1).reshape(B, 4)
    in_spec = pl.BlockSpec((_RB, V), lambda i: (i, 0))
    return pl.pallas_call(
        _tc_body,
        grid=(B // _RB,),
        in_specs=[in_spec, in_spec, in_spec,
                  pl.BlockSpec((_RB, 4), lambda i: (i, 0))],
        out_specs=pl.BlockSpec((2, _RB, V), lambda i: (0, i, 0)),
        out_shape=jax.ShapeDtypeStruct((2, B, V), jnp.float32),
    )(prior_bass_logits, prior_drums_logits, likelihood_logits, st)
